# baseline (speedup 1.0000x reference)
import functools

import jax
import jax.numpy as jnp
from jax import lax
from jax.experimental import pallas as pl
from jax.experimental.pallas import tpu as pltpu
from jax.experimental.pallas import tpu_sc as plsc

_NC = 2
_NS = 16
_CH = 32


def _make_sc_broadcast(B, H, dtype):
    topk = 2
    nw = _NC * _NS
    rows_per_w = B // nw
    n_chunks = rows_per_w // _CH
    mesh = plsc.VectorSubcoreMesh(core_axis_name="c", subcore_axis_name="s")

    @functools.partial(
        pl.kernel,
        mesh=mesh,
        out_type=jax.ShapeDtypeStruct((B, topk, H), dtype),
        scratch_types=[
            pltpu.VMEM((2, _CH, H), dtype),
            pltpu.SemaphoreType.DMA((2,)),
            pltpu.SemaphoreType.DMA((2,)),
        ],
    )
    def sc_broadcast(x_hbm, out_hbm, buf, rsem, wsem):
        wid = lax.axis_index("s") * _NC + lax.axis_index("c")
        base = wid * rows_per_w

        def read(c, slot):
            return pltpu.make_async_copy(
                x_hbm.at[pl.ds(base + c * _CH, _CH)], buf.at[slot], rsem.at[slot])

        def write(c, slot, k):
            return pltpu.make_async_copy(
                buf.at[slot], out_hbm.at[pl.ds(base + c * _CH, _CH), k],
                wsem.at[slot])

        read(0, 0).start()
        for c in range(n_chunks):
            s = c % 2
            read(c, s).wait()
            write(c, s, 0).start()
            write(c, s, 1).start()
            if c + 1 < n_chunks:
                if c >= 1:
                    write(c - 1, 1 - s, 0).wait()
                    write(c - 1, 1 - s, 1).wait()
                read(c + 1, 1 - s).start()
        write(n_chunks - 1, (n_chunks - 1) % 2, 0).wait()
        write(n_chunks - 1, (n_chunks - 1) % 2, 1).wait()

    return sc_broadcast


def kernel(x, expert_indices):
    del expert_indices
    B, H = x.shape
    return _make_sc_broadcast(B, H, x.dtype)(x)


# speedup vs baseline: 1.0379x; 1.0379x over previous
import functools

import jax
import jax.numpy as jnp
from jax import lax
from jax.experimental import pallas as pl
from jax.experimental.pallas import tpu as pltpu
from jax.experimental.pallas import tpu_sc as plsc

_NC = 2
_NS = 16
_CH = 16
_NSLOT = 3


def _make_sc_broadcast(B, H, dtype):
    topk = 2
    nw = _NC * _NS
    rows_per_w = B // nw
    n_chunks = rows_per_w // _CH
    mesh = plsc.VectorSubcoreMesh(core_axis_name="c", subcore_axis_name="s")

    @functools.partial(
        pl.kernel,
        mesh=mesh,
        out_type=jax.ShapeDtypeStruct((B, topk, H), dtype),
        scratch_types=[
            pltpu.VMEM((_NSLOT, _CH, H), dtype),
            pltpu.SemaphoreType.DMA((_NSLOT,)),
            pltpu.SemaphoreType.DMA((_NSLOT,)),
        ],
    )
    def sc_broadcast(x_hbm, out_hbm, buf, rsem, wsem):
        wid = lax.axis_index("s") * _NC + lax.axis_index("c")
        base = wid * rows_per_w

        def read(c, slot):
            return pltpu.make_async_copy(
                x_hbm.at[pl.ds(base + c * _CH, _CH)], buf.at[slot], rsem.at[slot])

        def write(c, slot, k):
            return pltpu.make_async_copy(
                buf.at[slot], out_hbm.at[pl.ds(base + c * _CH, _CH), k],
                wsem.at[slot])

        for c in range(min(_NSLOT, n_chunks)):
            read(c, c).start()
        for c in range(n_chunks):
            s = c % _NSLOT
            read(c, s).wait()
            write(c, s, 0).start()
            write(c, s, 1).start()
            m = c + _NSLOT
            if m < n_chunks:
                write(c, s, 0).wait()
                write(c, s, 1).wait()
                read(m, s).start()
        for c in range(max(0, n_chunks - _NSLOT), n_chunks):
            write(c, c % _NSLOT, 0).wait()
            write(c, c % _NSLOT, 1).wait()

    return sc_broadcast


def kernel(x, expert_indices):
    del expert_indices
    B, H = x.shape
    return _make_sc_broadcast(B, H, x.dtype)(x)


# speedup vs baseline: 1.5742x; 1.5168x over previous
import jax
import jax.numpy as jnp
from jax.experimental import pallas as pl

_BLK = 1024


def _broadcast_body(x_ref, o_ref):
    v = x_ref[...]
    o_ref[:, 0, :] = v
    o_ref[:, 1, :] = v


def kernel(x, expert_indices):
    del expert_indices
    B, H = x.shape
    topk = 2
    grid = (B // _BLK,)
    return pl.pallas_call(
        _broadcast_body,
        grid=grid,
        in_specs=[pl.BlockSpec((_BLK, H), lambda i: (i, 0))],
        out_specs=pl.BlockSpec((_BLK, topk, H), lambda i: (i, 0, 0)),
        out_shape=jax.ShapeDtypeStruct((B, topk, H), x.dtype),
    )(x)
